# Initial kernel scaffold; baseline (speedup 1.0000x reference)
import functools
import jax
import jax.numpy as jnp
from jax import lax
from jax.experimental import pallas as pl
from jax.experimental.pallas import tpu as pltpu, tpu_sc as plsc

VOCAB = 100000
LENGTH = 200
EMBED_DIM = 64
BATCH = 4096

_NW = 32
_N = BATCH * LENGTH
_ROWS_PER_W = _N // _NW
_SEQ_PER_CHUNK = 2
_CHUNK = _SEQ_PER_CHUNK * LENGTH
_NCHUNK = _ROWS_PER_W // _CHUNK


def _body(ids_hbm, table_hbm, pos_hbm, out_hbm, idx_v, dest_v, pos_v, sem):
    wid = lax.axis_index("c") * 16 + lax.axis_index("s")
    pltpu.sync_copy(pos_hbm, pos_v)

    def chunk_body(c, carry):
        base = wid * _ROWS_PER_W + c * _CHUNK
        pltpu.sync_copy(ids_hbm.at[pl.ds(base, _CHUNK)], idx_v)
        pltpu.async_copy(table_hbm.at[idx_v], dest_v, sem).wait()

        def add_body(i, acc):
            for j in range(EMBED_DIM // 16):
                sl = pl.ds(j * 16, 16)
                pv = pos_v[i, sl]
                for s in range(_SEQ_PER_CHUNK):
                    r = i + s * LENGTH
                    dest_v[r, sl] = dest_v[r, sl] + pv
            return acc

        lax.fori_loop(0, LENGTH, add_body, 0)
        pltpu.sync_copy(dest_v, out_hbm.at[pl.ds(base, _CHUNK)])
        return carry

    lax.fori_loop(0, _NCHUNK, chunk_body, 0)


@jax.jit
def _run(ids_flat, token_table, position_table):
    mesh = plsc.VectorSubcoreMesh(core_axis_name="c", subcore_axis_name="s")
    return pl.kernel(
        _body,
        out_type=jax.ShapeDtypeStruct((_N, EMBED_DIM), jnp.float32),
        mesh=mesh,
        scratch_types=[
            pltpu.VMEM((_CHUNK,), jnp.int32),
            pltpu.VMEM((_CHUNK, EMBED_DIM), jnp.float32),
            pltpu.VMEM((LENGTH, EMBED_DIM), jnp.float32),
            pltpu.SemaphoreType.DMA,
        ],
    )(ids_flat, token_table, position_table)


def kernel(input_ids, token_table, position_table):
    ids_flat = input_ids.reshape(_N).astype(jnp.int32)
    out = _run(ids_flat, token_table, position_table)
    return out.reshape(BATCH, LENGTH, EMBED_DIM)


# speedup vs baseline: 3.4609x; 3.4609x over previous
import functools
import jax
import jax.numpy as jnp
from jax import lax
from jax.experimental import pallas as pl
from jax.experimental.pallas import tpu as pltpu, tpu_sc as plsc

VOCAB = 100000
LENGTH = 200
EMBED_DIM = 64
BATCH = 4096

_NW = 32
_N = BATCH * LENGTH
_ROWS_PER_W = _N // _NW
_SEQ_PER_CHUNK = 2
_CHUNK = _SEQ_PER_CHUNK * LENGTH
_NCHUNK = _ROWS_PER_W // _CHUNK


def _body(ids_hbm, table_hbm, pos_hbm, out_hbm, idx_v, dest_v, pos_v, sem):
    wid = lax.axis_index("c") * 16 + lax.axis_index("s")
    pltpu.sync_copy(pos_hbm, pos_v)

    def chunk_body(c, carry):
        base = wid * _ROWS_PER_W + c * _CHUNK
        pltpu.sync_copy(ids_hbm.at[pl.ds(base, _CHUNK)], idx_v)
        pltpu.async_copy(table_hbm.at[idx_v], dest_v, sem).wait()

        def add_body(i, acc):
            for j in range(EMBED_DIM // 16):
                sl = pl.ds(j * 16, 16)
                pv = pos_v[i, sl]
                for s in range(_SEQ_PER_CHUNK):
                    r = i + s * LENGTH
                    dest_v[r, sl] = dest_v[r, sl] + pv
            return acc

        lax.fori_loop(0, LENGTH, add_body, 0)
        pltpu.sync_copy(dest_v, out_hbm.at[pl.ds(base, _CHUNK)])
        return carry

    lax.fori_loop(0, _NCHUNK, chunk_body, 0)


@jax.jit
def _run(ids_flat, token_table, position_table):
    mesh = plsc.VectorSubcoreMesh(core_axis_name="c", subcore_axis_name="s")
    return pl.kernel(
        _body,
        out_type=jax.ShapeDtypeStruct((_N, EMBED_DIM), jnp.float32),
        mesh=mesh,
        scratch_types=[
            pltpu.VMEM((_CHUNK,), jnp.int32),
            pltpu.VMEM((_CHUNK, EMBED_DIM), jnp.float32),
            pltpu.VMEM((LENGTH, EMBED_DIM), jnp.float32),
            pltpu.SemaphoreType.DMA,
        ],
        compiler_params=pltpu.CompilerParams(use_tc_tiling_on_sc=False),
    )(ids_flat, token_table, position_table)


def kernel(input_ids, token_table, position_table):
    ids_flat = input_ids.reshape(_N).astype(jnp.int32)
    out = _run(ids_flat, token_table, position_table)
    return out.reshape(BATCH, LENGTH, EMBED_DIM)


# speedup vs baseline: 4.0895x; 1.1816x over previous
import jax
import jax.numpy as jnp
from jax import lax
from jax.experimental import pallas as pl
from jax.experimental.pallas import tpu as pltpu, tpu_sc as plsc

VOCAB = 100000
LENGTH = 200
EMBED_DIM = 64
BATCH = 4096

_NW = 32
_N = BATCH * LENGTH
_ROWS_PER_W = _N // _NW
_SEQ_PER_CHUNK = 2
_CHUNK = _SEQ_PER_CHUNK * LENGTH
_NCHUNK = _ROWS_PER_W // _CHUNK
_NLANE = EMBED_DIM // 16


def _body(ids_hbm, table_hbm, pos_hbm, out_hbm,
          idx_v, dest_v, outb_v, pos_v, gsem0, gsem1, osem0, osem1):
    wid = lax.axis_index("c") * 16 + lax.axis_index("s")
    base0 = wid * _ROWS_PER_W
    gsems = (gsem0, gsem1)
    osems = (osem0, osem1)
    pltpu.sync_copy(pos_hbm, pos_v)

    def stage_and_gather(c, b):
        pltpu.sync_copy(ids_hbm.at[pl.ds(base0 + c * _CHUNK, _CHUNK)],
                        idx_v.at[b])
        pltpu.async_copy(table_hbm.at[idx_v.at[b]], dest_v.at[b], gsems[b])

    def wait_gather(b):
        pltpu.make_async_copy(table_hbm.at[idx_v.at[b]], dest_v.at[b],
                              gsems[b]).wait()

    def add_chunk(b):
        def add_body(i, acc):
            for j in range(_NLANE):
                sl = pl.ds(j * 16, 16)
                pv = pos_v[i, sl]
                for s in range(_SEQ_PER_CHUNK):
                    r = i + s * LENGTH
                    outb_v[b, r, sl] = dest_v[b, r, sl] + pv
            return acc
        lax.fori_loop(0, LENGTH, add_body, 0)

    def issue_out(c, b):
        pltpu.async_copy(outb_v.at[b],
                         out_hbm.at[pl.ds(base0 + c * _CHUNK, _CHUNK)],
                         osems[b])

    def wait_out(b):
        pltpu.make_async_copy(outb_v.at[b], out_hbm.at[pl.ds(0, _CHUNK)],
                              osems[b]).wait()

    stage_and_gather(0, 0)
    stage_and_gather(1, 1)

    for b in range(2):
        wait_gather(b)
        add_chunk(b)
        stage_and_gather(2 + b, b)
        issue_out(b, b)

    def pair_body(t, carry):
        for b in range(2):
            c = 2 * t + b
            wait_gather(b)
            wait_out(b)
            add_chunk(b)

            @pl.when(c + 2 < _NCHUNK)
            def _():
                stage_and_gather(c + 2, b)

            issue_out(c, b)
        return carry

    lax.fori_loop(1, _NCHUNK // 2, pair_body, 0)
    wait_out(0)
    wait_out(1)


@jax.jit
def _run(ids_flat, token_table, position_table):
    mesh = plsc.VectorSubcoreMesh(core_axis_name="c", subcore_axis_name="s")
    return pl.kernel(
        _body,
        out_type=jax.ShapeDtypeStruct((_N, EMBED_DIM), jnp.float32),
        mesh=mesh,
        scratch_types=[
            pltpu.VMEM((2, _CHUNK), jnp.int32),
            pltpu.VMEM((2, _CHUNK, EMBED_DIM), jnp.float32),
            pltpu.VMEM((2, _CHUNK, EMBED_DIM), jnp.float32),
            pltpu.VMEM((LENGTH, EMBED_DIM), jnp.float32),
            pltpu.SemaphoreType.DMA,
            pltpu.SemaphoreType.DMA,
            pltpu.SemaphoreType.DMA,
            pltpu.SemaphoreType.DMA,
        ],
        compiler_params=pltpu.CompilerParams(use_tc_tiling_on_sc=False),
    )(ids_flat, token_table, position_table)


def kernel(input_ids, token_table, position_table):
    ids_flat = input_ids.reshape(_N).astype(jnp.int32)
    out = _run(ids_flat, token_table, position_table)
    return out.reshape(BATCH, LENGTH, EMBED_DIM)
